# baseline (speedup 1.0000x reference)
import functools

import jax
import jax.numpy as jnp
from jax import lax
from jax.experimental import pallas as pl
from jax.experimental.pallas import tpu as pltpu
from jax.experimental.pallas import tpu_sc as plsc

KNN = 40
EPS = 1e-5
NEG_SLOPE = 0.2
N = 1024
B = 8
NWORK = 32
PPW = (B * N) // NWORK
OPAD = 128


def _dist_and_thresh(xb):
    G = lax.dot_general(xb, xb, (((1,), (1,)), ((), ())),
                        preferred_element_type=jnp.float32)
    xsq = jnp.sum(xb * xb, axis=1)
    inner = -2.0 * G
    D = ((-xsq)[:, None] - inner) - xsq[None, :]

    bits = lax.bitcast_convert_type(D, jnp.uint32)
    keys = jnp.where(bits >= jnp.uint32(0x80000000), ~bits,
                     bits | jnp.uint32(0x80000000))
    keysT = keys.T

    def body(i, prefix):
        bit = (jnp.uint32(31) - i.astype(jnp.uint32))
        cand = prefix | (jnp.uint32(1) << bit)
        cnt = jnp.sum((keysT >= cand[None, :]).astype(jnp.int32), axis=0)
        return jnp.where(cnt >= KNN, cand, prefix)

    prefix = lax.fori_loop(0, 32, body, jnp.zeros((N,), jnp.uint32))
    fb = jnp.where(prefix >= jnp.uint32(0x80000000),
                   prefix ^ jnp.uint32(0x80000000), ~prefix)
    thr = lax.bitcast_convert_type(fb, jnp.float32)
    return D - thr[:, None]


def _dist_x_kernel(x_ref, d_ref, xp_ref):
    xb = x_ref[0]
    d_ref[0] = _dist_and_thresh(xb)
    C = xb.shape[1]
    xp_ref[0] = jnp.concatenate(
        [xb, jnp.zeros((N, OPAD - C), jnp.float32)], axis=-1)


def _dist_kernel(x_ref, d_ref):
    xb = x_ref[0]
    d_ref[0] = _dist_and_thresh(xb)


def _run_dist_x(x):
    Cin = x.shape[-1]
    return pl.pallas_call(
        _dist_x_kernel,
        grid=(B,),
        in_specs=[pl.BlockSpec((1, N, Cin), lambda b: (b, 0, 0))],
        out_specs=[
            pl.BlockSpec((1, N, N), lambda b: (b, 0, 0)),
            pl.BlockSpec((1, N, OPAD), lambda b: (b, 0, 0)),
        ],
        out_shape=[
            jax.ShapeDtypeStruct((B, N, N), jnp.float32),
            jax.ShapeDtypeStruct((B, N, OPAD), jnp.float32),
        ],
    )(x)


def _run_dist(x):
    Cin = x.shape[-1]
    return pl.pallas_call(
        _dist_kernel,
        grid=(B,),
        in_specs=[pl.BlockSpec((1, N, Cin), lambda b: (b, 0, 0))],
        out_specs=pl.BlockSpec((1, N, N), lambda b: (b, 0, 0)),
        out_shape=jax.ShapeDtypeStruct((B, N, N), jnp.float32),
    )(x)


def _sc_compact(drow, selbuf, colbase, iota16, nchunks):
    def gt_body(j, off):
        d = drow[pl.ds(j * 16, 16)]
        m = d > 0.0
        idxv = (colbase + j * 16) + iota16
        plsc.store_compressed(selbuf.at[pl.ds(off, 16)], idxv, mask=m)
        return off + jnp.sum(m.astype(jnp.int32))

    off_gt = lax.fori_loop(0, nchunks, gt_body, jnp.int32(0))

    def eq_cond(c):
        j, off = c
        return jnp.logical_and(j < nchunks, off < KNN)

    def eq_body(c):
        j, off = c
        d = drow[pl.ds(j * 16, 16)]
        m = d == 0.0
        pos = jnp.full((16,), off, jnp.int32) \
            + plsc.cumsum(m.astype(jnp.int32)) - 1
        mc = jnp.logical_and(m, pos < KNN)
        idxv = (colbase + j * 16) + iota16
        plsc.store_compressed(selbuf.at[pl.ds(off, 16)], idxv, mask=mc)
        return j + 1, off + jnp.sum(mc.astype(jnp.int32))

    lax.while_loop(eq_cond, eq_body, (jnp.int32(0), off_gt))


def _sc_gather_rows(d_hbm, x_hbm, out_hbm, drow, selbuf, rows, outp, sem):
    wid = lax.axis_index("s") * 2 + lax.axis_index("c")
    base = wid * PPW
    colbase = (base // N) * N
    iota16 = lax.iota(jnp.int32, 16)
    nchunks = N // 16

    def point_body(p, carry):
        pltpu.sync_copy(d_hbm.at[base + p], drow)
        _sc_compact(drow, selbuf, colbase, iota16, nchunks)
        pltpu.async_copy(x_hbm.at[selbuf.at[pl.ds(0, KNN)]], rows,
                         sem).wait()

        def copy_body(k, c):
            outp[k] = rows[k, pl.ds(0, 16)]
            return c

        lax.fori_loop(0, KNN, copy_body, 0)
        pltpu.sync_copy(outp, out_hbm.at[pl.ds((base + p) * KNN, KNN)])
        return carry

    lax.fori_loop(0, PPW, point_body, 0)


def _run_sc_gather_rows(d_flat, xpad):
    mesh = plsc.VectorSubcoreMesh(core_axis_name="c", subcore_axis_name="s",
                                  num_cores=2, num_subcores=16)
    kfn = pl.kernel(
        _sc_gather_rows,
        out_type=jax.ShapeDtypeStruct((B * N * KNN, 16), jnp.float32),
        mesh=mesh,
        scratch_types=[
            pltpu.VMEM((N,), jnp.float32),
            pltpu.VMEM((KNN + 16,), jnp.int32),
            pltpu.VMEM((KNN, OPAD), jnp.float32),
            pltpu.VMEM((KNN, 16), jnp.float32),
            pltpu.SemaphoreType.DMA,
        ],
        compiler_params=pltpu.CompilerParams(needs_layout_passes=False),
    )
    return kfn(d_flat, xpad)


def _sc_gather_reduce(d_hbm, u_hbm, out_hbm,
                      drow, selbuf, rows, outbuf, sem):
    wid = lax.axis_index("s") * 2 + lax.axis_index("c")
    base = wid * PPW
    colbase = (base // N) * N
    iota16 = lax.iota(jnp.int32, 16)
    nchunks = N // 16

    def point_body(p, carry):
        pltpu.sync_copy(d_hbm.at[base + p], drow)
        _sc_compact(drow, selbuf, colbase, iota16, nchunks)

        pltpu.async_copy(u_hbm.at[selbuf.at[pl.ds(0, KNN)]], rows,
                         sem).wait()

        ninf = jnp.full((16,), -jnp.inf, jnp.float32)
        pinf = jnp.full((16,), jnp.inf, jnp.float32)
        zero = jnp.zeros((16,), jnp.float32)
        acc0 = (ninf, ninf, ninf, ninf, pinf, pinf, pinf, pinf,
                zero, zero, zero, zero, zero, zero, zero, zero)

        def red_body(k, acc):
            out = list(acc)
            for c in range(4):
                rv = rows[k, pl.ds(c * 16, 16)]
                out[c] = jnp.maximum(out[c], rv)
                out[4 + c] = jnp.minimum(out[4 + c], rv)
                out[8 + c] = out[8 + c] + rv
                out[12 + c] = out[12 + c] + rv * rv
            return tuple(out)

        acc = lax.fori_loop(0, KNN, red_body, acc0)
        for c in range(4):
            outbuf[p, pl.ds(c * 16, 16)] = acc[c]
            outbuf[p, pl.ds(64 + c * 16, 16)] = acc[4 + c]
            outbuf[p, pl.ds(128 + c * 16, 16)] = acc[8 + c]
            outbuf[p, pl.ds(192 + c * 16, 16)] = acc[12 + c]
        return carry

    lax.fori_loop(0, PPW, point_body, 0)
    pltpu.sync_copy(outbuf, out_hbm.at[pl.ds(base, PPW)])


def _run_sc_gather(d_flat, u_flat, O=64):
    mesh = plsc.VectorSubcoreMesh(core_axis_name="c", subcore_axis_name="s",
                                  num_cores=2, num_subcores=16)
    kfn = pl.kernel(
        _sc_gather_reduce,
        out_type=jax.ShapeDtypeStruct((B * N, 4 * O), jnp.float32),
        mesh=mesh,
        scratch_types=[
            pltpu.VMEM((N,), jnp.float32),
            pltpu.VMEM((KNN + 16,), jnp.int32),
            pltpu.VMEM((KNN, OPAD), jnp.float32),
            pltpu.VMEM((PPW, 4 * O), jnp.float32),
            pltpu.SemaphoreType.DMA,
        ],
        compiler_params=pltpu.CompilerParams(needs_layout_passes=False),
    )
    return kfn(d_flat, u_flat)


PB = 64
NBLK = (B * N) // PB


def _edgey_kernel(xg_ref, xf_ref, w_ref, ymax_ref, ymin_ref,
                  psum_ref, psq_ref):
    C = xf_ref.shape[-1]
    xj = xg_ref[:, 0:C]
    xn = jnp.broadcast_to(xf_ref[...][:, None, :],
                          (PB, KNN, C)).reshape(PB * KNN, C)
    edge = jnp.concatenate([xj - xn, xn], axis=-1)
    y = lax.dot_general(edge, w_ref[...], (((1,), (1,)), ((), ())),
                        preferred_element_type=jnp.float32)
    O = y.shape[-1]
    y3 = y.reshape(PB, KNN, O)
    ymax_ref[...] = jnp.max(y3, axis=1)
    ymin_ref[...] = jnp.min(y3, axis=1)
    psum_ref[...] = jnp.sum(y, axis=0)[None, None]
    psq_ref[...] = jnp.sum(y * y, axis=0)[None, None]


def _run_edgey(xg, x_flat, W):
    C = x_flat.shape[-1]
    O = W.shape[0]
    return pl.pallas_call(
        _edgey_kernel,
        grid=(NBLK,),
        in_specs=[
            pl.BlockSpec((PB * KNN, 16), lambda i: (i, 0)),
            pl.BlockSpec((PB, C), lambda i: (i, 0)),
            pl.BlockSpec(W.shape, lambda i: (0, 0)),
        ],
        out_specs=[
            pl.BlockSpec((PB, O), lambda i: (i, 0)),
            pl.BlockSpec((PB, O), lambda i: (i, 0)),
            pl.BlockSpec((1, 1, O), lambda i: (i, 0, 0)),
            pl.BlockSpec((1, 1, O), lambda i: (i, 0, 0)),
        ],
        out_shape=[
            jax.ShapeDtypeStruct((B * N, O), jnp.float32),
            jax.ShapeDtypeStruct((B * N, O), jnp.float32),
            jax.ShapeDtypeStruct((NBLK, 1, O), jnp.float32),
            jax.ShapeDtypeStruct((NBLK, 1, O), jnp.float32),
        ],
    )(xg, x_flat, W)


def _bn_finalize_kernel(ymax_ref, ymin_ref, psum_ref, psq_ref,
                        g_ref, be_ref, w2_ref, h1_ref, u2_ref, v2_ref):
    denom = float(B * N * KNN)
    mean = jnp.sum(psum_ref[...][:, 0, :], axis=0) / denom
    ey2 = jnp.sum(psq_ref[...][:, 0, :], axis=0) / denom
    var = ey2 - mean * mean
    s = jnp.sqrt(var + EPS)
    g = g_ref[...]
    be = be_ref[...]
    M = jnp.where((g >= 0)[None, :], ymax_ref[...], ymin_ref[...])
    yn = (g[None, :] * (M - mean[None, :])) / s[None, :] + be[None, :]
    h1 = jnp.where(yn >= 0, yn, NEG_SLOPE * yn)
    h1_ref[...] = h1
    C = h1.shape[-1]
    wd = w2_ref[:, :C]
    wc = w2_ref[:, C:]
    u2 = lax.dot_general(h1, wd, (((1,), (1,)), ((), ())),
                         preferred_element_type=jnp.float32)
    O2 = u2.shape[-1]
    u2_ref[...] = jnp.concatenate(
        [u2, jnp.zeros((B * N, OPAD - O2), jnp.float32)], axis=-1)
    v2_ref[...] = lax.dot_general(h1, wc - wd, (((1,), (1,)), ((), ())),
                                  preferred_element_type=jnp.float32)


def _run_bn_finalize(ymax, ymin, psum, psq, g, be, W2):
    O = ymax.shape[-1]
    O2 = W2.shape[0]
    return pl.pallas_call(
        _bn_finalize_kernel,
        out_shape=[
            jax.ShapeDtypeStruct((B * N, O), jnp.float32),
            jax.ShapeDtypeStruct((B * N, OPAD), jnp.float32),
            jax.ShapeDtypeStruct((B * N, O2), jnp.float32),
        ],
    )(ymax, ymin, psum, psq, g, be, W2)


def _bn_stats_kernel(gstats_ref, vv_ref, mean_ref, rstd_ref):
    O = vv_ref.shape[-1]
    gsum = gstats_ref[:, 2 * O:3 * O]
    gsq = gstats_ref[:, 3 * O:4 * O]
    v = vv_ref[...]
    denom = float(B * N * KNN)
    mean = (jnp.sum(gsum, axis=0) + KNN * jnp.sum(v, axis=0)) / denom
    ey2 = (jnp.sum(gsq, axis=0) + 2.0 * jnp.sum(v * gsum, axis=0)
           + KNN * jnp.sum(v * v, axis=0)) / denom
    var = ey2 - mean * mean
    mean_ref[...] = mean
    rstd_ref[...] = lax.rsqrt(var + EPS)


def _run_bn_stats(gstats, v_flat):
    O = v_flat.shape[-1]
    return pl.pallas_call(
        _bn_stats_kernel,
        out_shape=[
            jax.ShapeDtypeStruct((O,), jnp.float32),
            jax.ShapeDtypeStruct((O,), jnp.float32),
        ],
    )(gstats, v_flat)


def _final_kernel(h1_ref, gstats_ref, v2_ref, mean_ref, rstd_ref,
                  g_ref, be_ref, wf_ref, bf_ref, out_ref):
    O = v2_ref.shape[-1]
    gmax = gstats_ref[0, :, 0:O]
    gmin = gstats_ref[0, :, O:2 * O]
    v = v2_ref[0]
    g = g_ref[...]
    be = be_ref[...]
    M = jnp.where((g >= 0)[None, :], gmax + v, gmin + v)
    yn = g[None, :] * (M - mean_ref[...][None, :]) * rstd_ref[...][None, :] \
        + be[None, :]
    h2 = jnp.where(yn >= 0, yn, NEG_SLOPE * yn)
    hcat = jnp.concatenate([h1_ref[0], h2], axis=-1)
    o = jnp.dot(hcat, wf_ref[...], preferred_element_type=jnp.float32)
    o = o + bf_ref[...][None, :]
    out_ref[0, 0] = jnp.max(o, axis=0)


def _run_final(h1, gstats2, v2, mean2, rstd2, g2, be2, Wf, bf):
    O = v2.shape[-1]
    Z = Wf.shape[1]
    return pl.pallas_call(
        _final_kernel,
        grid=(B,),
        in_specs=[
            pl.BlockSpec((1, N, O), lambda b: (b, 0, 0)),
            pl.BlockSpec((1, N, 4 * O), lambda b: (b, 0, 0)),
            pl.BlockSpec((1, N, O), lambda b: (b, 0, 0)),
            pl.BlockSpec((O,), lambda b: (0,)),
            pl.BlockSpec((O,), lambda b: (0,)),
            pl.BlockSpec((O,), lambda b: (0,)),
            pl.BlockSpec((O,), lambda b: (0,)),
            pl.BlockSpec(Wf.shape, lambda b: (0, 0)),
            pl.BlockSpec((Z,), lambda b: (0,)),
        ],
        out_specs=pl.BlockSpec((1, 1, Z), lambda b: (b, 0, 0)),
        out_shape=jax.ShapeDtypeStruct((B, 1, Z), jnp.float32),
    )(h1, gstats2, v2, mean2, rstd2, g2, be2, Wf, bf)


def kernel(x, W1, b1, g1, be1, W2, b2, g2, be2, Wf, bf):
    D1, xpad = _run_dist_x(x)
    xg = _run_sc_gather_rows(D1.reshape(B * N, N),
                             xpad.reshape(B * N, OPAD))
    ymax, ymin, psum, psq = _run_edgey(xg, x.reshape(B * N, -1), W1)
    h1f, u2f, v2f = _run_bn_finalize(ymax, ymin, psum, psq, g1, be1, W2)
    h1 = h1f.reshape(B, N, -1)
    D2 = _run_dist(h1)
    gstats2 = _run_sc_gather(D2.reshape(B * N, N), u2f)
    mean2, rstd2 = _run_bn_stats(gstats2, v2f)
    out = _run_final(h1, gstats2.reshape(B, N, -1), v2f.reshape(B, N, -1),
                     mean2, rstd2, g2, be2, Wf, bf)
    return out[:, 0, :]


# speedup vs baseline: 1.1590x; 1.1590x over previous
import functools

import jax
import jax.numpy as jnp
from jax import lax
from jax.experimental import pallas as pl
from jax.experimental.pallas import tpu as pltpu
from jax.experimental.pallas import tpu_sc as plsc

KNN = 40
EPS = 1e-5
NEG_SLOPE = 0.2
N = 1024
B = 8
NWORK = 32
PPW = (B * N) // NWORK
OPAD = 128


def _dist_and_thresh(xb):
    G = lax.dot_general(xb, xb, (((1,), (1,)), ((), ())),
                        preferred_element_type=jnp.float32)
    xsq = jnp.sum(xb * xb, axis=1)
    inner = -2.0 * G
    D = ((-xsq)[:, None] - inner) - xsq[None, :]

    bits = lax.bitcast_convert_type(D, jnp.uint32)
    keys = jnp.where(bits >= jnp.uint32(0x80000000), ~bits,
                     bits | jnp.uint32(0x80000000))
    keysT = keys.T

    def body(i, prefix):
        bit = (jnp.uint32(31) - i.astype(jnp.uint32))
        cand = prefix | (jnp.uint32(1) << bit)
        cnt = jnp.sum((keysT >= cand[None, :]).astype(jnp.int32), axis=0)
        return jnp.where(cnt >= KNN, cand, prefix)

    prefix = lax.fori_loop(0, 32, body, jnp.zeros((N,), jnp.uint32))
    fb = jnp.where(prefix >= jnp.uint32(0x80000000),
                   prefix ^ jnp.uint32(0x80000000), ~prefix)
    thr = lax.bitcast_convert_type(fb, jnp.float32)
    return D - thr[:, None]


def _dist_x_kernel(x_ref, d_ref, xp_ref):
    xb = x_ref[0]
    d_ref[0] = _dist_and_thresh(xb)
    C = xb.shape[1]
    xp_ref[0] = jnp.concatenate(
        [xb, jnp.zeros((N, OPAD - C), jnp.float32)], axis=-1)


def _dist_kernel(x_ref, d_ref):
    xb = x_ref[0]
    d_ref[0] = _dist_and_thresh(xb)


def _run_dist_x(x):
    Cin = x.shape[-1]
    return pl.pallas_call(
        _dist_x_kernel,
        grid=(B,),
        in_specs=[pl.BlockSpec((1, N, Cin), lambda b: (b, 0, 0))],
        out_specs=[
            pl.BlockSpec((1, N, N), lambda b: (b, 0, 0)),
            pl.BlockSpec((1, N, OPAD), lambda b: (b, 0, 0)),
        ],
        out_shape=[
            jax.ShapeDtypeStruct((B, N, N), jnp.float32),
            jax.ShapeDtypeStruct((B, N, OPAD), jnp.float32),
        ],
    )(x)


def _run_dist(x):
    Cin = x.shape[-1]
    return pl.pallas_call(
        _dist_kernel,
        grid=(B,),
        in_specs=[pl.BlockSpec((1, N, Cin), lambda b: (b, 0, 0))],
        out_specs=pl.BlockSpec((1, N, N), lambda b: (b, 0, 0)),
        out_shape=jax.ShapeDtypeStruct((B, N, N), jnp.float32),
    )(x)


def _sc_compact(drow, selbuf, colbase, iota16, nchunks):
    def gt_body(j, off):
        d = drow[pl.ds(j * 16, 16)]
        m = d > 0.0
        idxv = (colbase + j * 16) + iota16
        plsc.store_compressed(selbuf.at[pl.ds(off, 16)], idxv, mask=m)
        return off + jnp.sum(m.astype(jnp.int32))

    off_gt = lax.fori_loop(0, nchunks, gt_body, jnp.int32(0))

    def eq_cond(c):
        j, off = c
        return jnp.logical_and(j < nchunks, off < KNN)

    def eq_body(c):
        j, off = c
        d = drow[pl.ds(j * 16, 16)]
        m = d == 0.0
        pos = jnp.full((16,), off, jnp.int32) \
            + plsc.cumsum(m.astype(jnp.int32)) - 1
        mc = jnp.logical_and(m, pos < KNN)
        idxv = (colbase + j * 16) + iota16
        plsc.store_compressed(selbuf.at[pl.ds(off, 16)], idxv, mask=mc)
        return j + 1, off + jnp.sum(mc.astype(jnp.int32))

    lax.while_loop(eq_cond, eq_body, (jnp.int32(0), off_gt))


def _pipelined_points(d_hbm, drow0, drow1, sem0, sem1, base, process):
    pltpu.async_copy(d_hbm.at[base], drow0, sem0)

    def pair_body(i, carry):
        p0 = 2 * i
        pltpu.make_async_copy(d_hbm.at[base], drow0, sem0).wait()
        pltpu.async_copy(d_hbm.at[base + p0 + 1], drow1, sem1)
        process(drow0, p0)
        pltpu.make_async_copy(d_hbm.at[base], drow1, sem1).wait()
        nxt = jnp.minimum(base + p0 + 2, B * N - 1)
        pltpu.async_copy(d_hbm.at[nxt], drow0, sem0)
        process(drow1, p0 + 1)
        return carry

    lax.fori_loop(0, PPW // 2, pair_body, 0)
    pltpu.make_async_copy(d_hbm.at[base], drow0, sem0).wait()


def _sc_gather_rows(d_hbm, x_hbm, out_hbm, drow0, drow1, selbuf, rows,
                    outp, sem0, sem1, gsem):
    wid = lax.axis_index("s") * 2 + lax.axis_index("c")
    base = wid * PPW
    colbase = (base // N) * N
    iota16 = lax.iota(jnp.int32, 16)
    nchunks = N // 16

    def process(drow, p):
        _sc_compact(drow, selbuf, colbase, iota16, nchunks)
        pltpu.async_copy(x_hbm.at[selbuf.at[pl.ds(0, KNN)]], rows,
                         gsem).wait()

        def copy_body(k, c):
            outp[k] = rows[k, pl.ds(0, 16)]
            return c

        lax.fori_loop(0, KNN, copy_body, 0)
        pltpu.sync_copy(outp, out_hbm.at[pl.ds((base + p) * KNN, KNN)])

    _pipelined_points(d_hbm, drow0, drow1, sem0, sem1, base, process)


def _run_sc_gather_rows(d_flat, xpad):
    mesh = plsc.VectorSubcoreMesh(core_axis_name="c", subcore_axis_name="s",
                                  num_cores=2, num_subcores=16)
    kfn = pl.kernel(
        _sc_gather_rows,
        out_type=jax.ShapeDtypeStruct((B * N * KNN, 16), jnp.float32),
        mesh=mesh,
        scratch_types=[
            pltpu.VMEM((N,), jnp.float32),
            pltpu.VMEM((N,), jnp.float32),
            pltpu.VMEM((KNN + 16,), jnp.int32),
            pltpu.VMEM((KNN, OPAD), jnp.float32),
            pltpu.VMEM((KNN, 16), jnp.float32),
            pltpu.SemaphoreType.DMA,
            pltpu.SemaphoreType.DMA,
            pltpu.SemaphoreType.DMA,
        ],
        compiler_params=pltpu.CompilerParams(needs_layout_passes=False),
    )
    return kfn(d_flat, xpad)


def _sc_gather_reduce(d_hbm, u_hbm, out_hbm,
                      drow0, drow1, selbuf, rows, outbuf,
                      sem0, sem1, gsem):
    wid = lax.axis_index("s") * 2 + lax.axis_index("c")
    base = wid * PPW
    colbase = (base // N) * N
    iota16 = lax.iota(jnp.int32, 16)
    nchunks = N // 16

    def process(drow, p):
        _sc_compact(drow, selbuf, colbase, iota16, nchunks)

        pltpu.async_copy(u_hbm.at[selbuf.at[pl.ds(0, KNN)]], rows,
                         gsem).wait()

        ninf = jnp.full((16,), -jnp.inf, jnp.float32)
        pinf = jnp.full((16,), jnp.inf, jnp.float32)
        zero = jnp.zeros((16,), jnp.float32)
        acc0 = (ninf, ninf, ninf, ninf, pinf, pinf, pinf, pinf,
                zero, zero, zero, zero, zero, zero, zero, zero)

        def red_body(k, acc):
            out = list(acc)
            for c in range(4):
                rv = rows[k, pl.ds(c * 16, 16)]
                out[c] = jnp.maximum(out[c], rv)
                out[4 + c] = jnp.minimum(out[4 + c], rv)
                out[8 + c] = out[8 + c] + rv
                out[12 + c] = out[12 + c] + rv * rv
            return tuple(out)

        acc = lax.fori_loop(0, KNN, red_body, acc0)
        for c in range(4):
            outbuf[p, pl.ds(c * 16, 16)] = acc[c]
            outbuf[p, pl.ds(64 + c * 16, 16)] = acc[4 + c]
            outbuf[p, pl.ds(128 + c * 16, 16)] = acc[8 + c]
            outbuf[p, pl.ds(192 + c * 16, 16)] = acc[12 + c]

    _pipelined_points(d_hbm, drow0, drow1, sem0, sem1, base, process)
    pltpu.sync_copy(outbuf, out_hbm.at[pl.ds(base, PPW)])


def _run_sc_gather(d_flat, u_flat, O=64):
    mesh = plsc.VectorSubcoreMesh(core_axis_name="c", subcore_axis_name="s",
                                  num_cores=2, num_subcores=16)
    kfn = pl.kernel(
        _sc_gather_reduce,
        out_type=jax.ShapeDtypeStruct((B * N, 4 * O), jnp.float32),
        mesh=mesh,
        scratch_types=[
            pltpu.VMEM((N,), jnp.float32),
            pltpu.VMEM((N,), jnp.float32),
            pltpu.VMEM((KNN + 16,), jnp.int32),
            pltpu.VMEM((KNN, OPAD), jnp.float32),
            pltpu.VMEM((PPW, 4 * O), jnp.float32),
            pltpu.SemaphoreType.DMA,
            pltpu.SemaphoreType.DMA,
            pltpu.SemaphoreType.DMA,
        ],
        compiler_params=pltpu.CompilerParams(needs_layout_passes=False),
    )
    return kfn(d_flat, u_flat)


PB = 64
NBLK = (B * N) // PB


def _edgey_kernel(xg_ref, xf_ref, w_ref, ymax_ref, ymin_ref,
                  psum_ref, psq_ref):
    C = xf_ref.shape[-1]
    xj = xg_ref[:, 0:C]
    xn = jnp.broadcast_to(xf_ref[...][:, None, :],
                          (PB, KNN, C)).reshape(PB * KNN, C)
    edge = jnp.concatenate([xj - xn, xn], axis=-1)
    y = lax.dot_general(edge, w_ref[...], (((1,), (1,)), ((), ())),
                        preferred_element_type=jnp.float32)
    O = y.shape[-1]
    y3 = y.reshape(PB, KNN, O)
    ymax_ref[...] = jnp.max(y3, axis=1)
    ymin_ref[...] = jnp.min(y3, axis=1)
    psum_ref[...] = jnp.sum(y, axis=0)[None, None]
    psq_ref[...] = jnp.sum(y * y, axis=0)[None, None]


def _run_edgey(xg, x_flat, W):
    C = x_flat.shape[-1]
    O = W.shape[0]
    return pl.pallas_call(
        _edgey_kernel,
        grid=(NBLK,),
        in_specs=[
            pl.BlockSpec((PB * KNN, 16), lambda i: (i, 0)),
            pl.BlockSpec((PB, C), lambda i: (i, 0)),
            pl.BlockSpec(W.shape, lambda i: (0, 0)),
        ],
        out_specs=[
            pl.BlockSpec((PB, O), lambda i: (i, 0)),
            pl.BlockSpec((PB, O), lambda i: (i, 0)),
            pl.BlockSpec((1, 1, O), lambda i: (i, 0, 0)),
            pl.BlockSpec((1, 1, O), lambda i: (i, 0, 0)),
        ],
        out_shape=[
            jax.ShapeDtypeStruct((B * N, O), jnp.float32),
            jax.ShapeDtypeStruct((B * N, O), jnp.float32),
            jax.ShapeDtypeStruct((NBLK, 1, O), jnp.float32),
            jax.ShapeDtypeStruct((NBLK, 1, O), jnp.float32),
        ],
    )(xg, x_flat, W)


def _bn_finalize_kernel(ymax_ref, ymin_ref, psum_ref, psq_ref,
                        g_ref, be_ref, w2_ref, h1_ref, u2_ref, v2_ref):
    denom = float(B * N * KNN)
    mean = jnp.sum(psum_ref[...][:, 0, :], axis=0) / denom
    ey2 = jnp.sum(psq_ref[...][:, 0, :], axis=0) / denom
    var = ey2 - mean * mean
    s = jnp.sqrt(var + EPS)
    g = g_ref[...]
    be = be_ref[...]
    M = jnp.where((g >= 0)[None, :], ymax_ref[...], ymin_ref[...])
    yn = (g[None, :] * (M - mean[None, :])) / s[None, :] + be[None, :]
    h1 = jnp.where(yn >= 0, yn, NEG_SLOPE * yn)
    h1_ref[...] = h1
    C = h1.shape[-1]
    wd = w2_ref[:, :C]
    wc = w2_ref[:, C:]
    u2 = lax.dot_general(h1, wd, (((1,), (1,)), ((), ())),
                         preferred_element_type=jnp.float32)
    O2 = u2.shape[-1]
    u2_ref[...] = jnp.concatenate(
        [u2, jnp.zeros((B * N, OPAD - O2), jnp.float32)], axis=-1)
    v2_ref[...] = lax.dot_general(h1, wc - wd, (((1,), (1,)), ((), ())),
                                  preferred_element_type=jnp.float32)


def _run_bn_finalize(ymax, ymin, psum, psq, g, be, W2):
    O = ymax.shape[-1]
    O2 = W2.shape[0]
    return pl.pallas_call(
        _bn_finalize_kernel,
        out_shape=[
            jax.ShapeDtypeStruct((B * N, O), jnp.float32),
            jax.ShapeDtypeStruct((B * N, OPAD), jnp.float32),
            jax.ShapeDtypeStruct((B * N, O2), jnp.float32),
        ],
    )(ymax, ymin, psum, psq, g, be, W2)


def _bn_stats_kernel(gstats_ref, vv_ref, mean_ref, rstd_ref):
    O = vv_ref.shape[-1]
    gsum = gstats_ref[:, 2 * O:3 * O]
    gsq = gstats_ref[:, 3 * O:4 * O]
    v = vv_ref[...]
    denom = float(B * N * KNN)
    mean = (jnp.sum(gsum, axis=0) + KNN * jnp.sum(v, axis=0)) / denom
    ey2 = (jnp.sum(gsq, axis=0) + 2.0 * jnp.sum(v * gsum, axis=0)
           + KNN * jnp.sum(v * v, axis=0)) / denom
    var = ey2 - mean * mean
    mean_ref[...] = mean
    rstd_ref[...] = lax.rsqrt(var + EPS)


def _run_bn_stats(gstats, v_flat):
    O = v_flat.shape[-1]
    return pl.pallas_call(
        _bn_stats_kernel,
        out_shape=[
            jax.ShapeDtypeStruct((O,), jnp.float32),
            jax.ShapeDtypeStruct((O,), jnp.float32),
        ],
    )(gstats, v_flat)


def _final_kernel(h1_ref, gstats_ref, v2_ref, mean_ref, rstd_ref,
                  g_ref, be_ref, wf_ref, bf_ref, out_ref):
    O = v2_ref.shape[-1]
    gmax = gstats_ref[0, :, 0:O]
    gmin = gstats_ref[0, :, O:2 * O]
    v = v2_ref[0]
    g = g_ref[...]
    be = be_ref[...]
    M = jnp.where((g >= 0)[None, :], gmax + v, gmin + v)
    yn = g[None, :] * (M - mean_ref[...][None, :]) * rstd_ref[...][None, :] \
        + be[None, :]
    h2 = jnp.where(yn >= 0, yn, NEG_SLOPE * yn)
    hcat = jnp.concatenate([h1_ref[0], h2], axis=-1)
    o = jnp.dot(hcat, wf_ref[...], preferred_element_type=jnp.float32)
    o = o + bf_ref[...][None, :]
    out_ref[0, 0] = jnp.max(o, axis=0)


def _run_final(h1, gstats2, v2, mean2, rstd2, g2, be2, Wf, bf):
    O = v2.shape[-1]
    Z = Wf.shape[1]
    return pl.pallas_call(
        _final_kernel,
        grid=(B,),
        in_specs=[
            pl.BlockSpec((1, N, O), lambda b: (b, 0, 0)),
            pl.BlockSpec((1, N, 4 * O), lambda b: (b, 0, 0)),
            pl.BlockSpec((1, N, O), lambda b: (b, 0, 0)),
            pl.BlockSpec((O,), lambda b: (0,)),
            pl.BlockSpec((O,), lambda b: (0,)),
            pl.BlockSpec((O,), lambda b: (0,)),
            pl.BlockSpec((O,), lambda b: (0,)),
            pl.BlockSpec(Wf.shape, lambda b: (0, 0)),
            pl.BlockSpec((Z,), lambda b: (0,)),
        ],
        out_specs=pl.BlockSpec((1, 1, Z), lambda b: (b, 0, 0)),
        out_shape=jax.ShapeDtypeStruct((B, 1, Z), jnp.float32),
    )(h1, gstats2, v2, mean2, rstd2, g2, be2, Wf, bf)


def kernel(x, W1, b1, g1, be1, W2, b2, g2, be2, Wf, bf):
    D1, xpad = _run_dist_x(x)
    xg = _run_sc_gather_rows(D1.reshape(B * N, N),
                             xpad.reshape(B * N, OPAD))
    ymax, ymin, psum, psq = _run_edgey(xg, x.reshape(B * N, -1), W1)
    h1f, u2f, v2f = _run_bn_finalize(ymax, ymin, psum, psq, g1, be1, W2)
    h1 = h1f.reshape(B, N, -1)
    D2 = _run_dist(h1)
    gstats2 = _run_sc_gather(D2.reshape(B * N, N), u2f)
    mean2, rstd2 = _run_bn_stats(gstats2, v2f)
    out = _run_final(h1, gstats2.reshape(B, N, -1), v2f.reshape(B, N, -1),
                     mean2, rstd2, g2, be2, Wf, bf)
    return out[:, 0, :]


# speedup vs baseline: 1.1973x; 1.0330x over previous
import functools

import jax
import jax.numpy as jnp
from jax import lax
from jax.experimental import pallas as pl
from jax.experimental.pallas import tpu as pltpu
from jax.experimental.pallas import tpu_sc as plsc

KNN = 40
EPS = 1e-5
NEG_SLOPE = 0.2
N = 1024
B = 8
NWORK = 32
PPW = (B * N) // NWORK
OPAD = 128


def _dist_and_thresh(xb):
    G = lax.dot_general(xb, xb, (((1,), (1,)), ((), ())),
                        preferred_element_type=jnp.float32)
    xsq = jnp.sum(xb * xb, axis=1)
    inner = -2.0 * G
    D = ((-xsq)[:, None] - inner) - xsq[None, :]

    bits = lax.bitcast_convert_type(D, jnp.uint32)
    keys = jnp.where(bits >= jnp.uint32(0x80000000), ~bits,
                     bits | jnp.uint32(0x80000000))
    keysT = keys.T

    def body(i, prefix):
        bit = (jnp.uint32(31) - i.astype(jnp.uint32))
        cand = prefix | (jnp.uint32(1) << bit)
        cnt = jnp.sum((keysT >= cand[None, :]).astype(jnp.int32), axis=0)
        return jnp.where(cnt >= KNN, cand, prefix)

    prefix = lax.fori_loop(0, 32, body, jnp.zeros((N,), jnp.uint32))
    fb = jnp.where(prefix >= jnp.uint32(0x80000000),
                   prefix ^ jnp.uint32(0x80000000), ~prefix)
    thr = lax.bitcast_convert_type(fb, jnp.float32)
    return D - thr[:, None]


def _dist_x_kernel(x_ref, d_ref, xp_ref):
    xb = x_ref[0]
    d_ref[0] = _dist_and_thresh(xb)
    C = xb.shape[1]
    xp_ref[0] = jnp.concatenate(
        [xb, jnp.zeros((N, OPAD - C), jnp.float32)], axis=-1)


def _dist_kernel(x_ref, d_ref):
    xb = x_ref[0]
    d_ref[0] = _dist_and_thresh(xb)


def _run_dist_x(x):
    Cin = x.shape[-1]
    return pl.pallas_call(
        _dist_x_kernel,
        grid=(B,),
        in_specs=[pl.BlockSpec((1, N, Cin), lambda b: (b, 0, 0))],
        out_specs=[
            pl.BlockSpec((1, N, N), lambda b: (b, 0, 0)),
            pl.BlockSpec((1, N, OPAD), lambda b: (b, 0, 0)),
        ],
        out_shape=[
            jax.ShapeDtypeStruct((B, N, N), jnp.float32),
            jax.ShapeDtypeStruct((B, N, OPAD), jnp.float32),
        ],
    )(x)


def _run_dist(x):
    Cin = x.shape[-1]
    return pl.pallas_call(
        _dist_kernel,
        grid=(B,),
        in_specs=[pl.BlockSpec((1, N, Cin), lambda b: (b, 0, 0))],
        out_specs=pl.BlockSpec((1, N, N), lambda b: (b, 0, 0)),
        out_shape=jax.ShapeDtypeStruct((B, N, N), jnp.float32),
    )(x)


def _sc_compact(drow, selbuf, colbase, iota16, nchunks):
    def gt_body(j, off):
        d = drow[pl.ds(j * 16, 16)]
        m = d > 0.0
        idxv = (colbase + j * 16) + iota16
        plsc.store_compressed(selbuf.at[pl.ds(off, 16)], idxv, mask=m)
        return off + jnp.sum(m.astype(jnp.int32))

    off_gt = lax.fori_loop(0, nchunks, gt_body, jnp.int32(0))

    def eq_cond(c):
        j, off = c
        return jnp.logical_and(j < nchunks, off < KNN)

    def eq_body(c):
        j, off = c
        d = drow[pl.ds(j * 16, 16)]
        m = d == 0.0
        pos = jnp.full((16,), off, jnp.int32) \
            + plsc.cumsum(m.astype(jnp.int32)) - 1
        mc = jnp.logical_and(m, pos < KNN)
        idxv = (colbase + j * 16) + iota16
        plsc.store_compressed(selbuf.at[pl.ds(off, 16)], idxv, mask=mc)
        return j + 1, off + jnp.sum(mc.astype(jnp.int32))

    lax.while_loop(eq_cond, eq_body, (jnp.int32(0), off_gt))


def _pipelined_points2(d_hbm, drow0, drow1, sem0, sem1, base,
                       process_even, process_odd):
    pltpu.async_copy(d_hbm.at[base], drow0, sem0)

    def pair_body(i, carry):
        p0 = 2 * i
        pltpu.make_async_copy(d_hbm.at[base], drow0, sem0).wait()
        pltpu.async_copy(d_hbm.at[base + p0 + 1], drow1, sem1)
        process_even(drow0, p0)
        pltpu.make_async_copy(d_hbm.at[base], drow1, sem1).wait()
        nxt = jnp.minimum(base + p0 + 2, B * N - 1)
        pltpu.async_copy(d_hbm.at[nxt], drow0, sem0)
        process_odd(drow1, p0 + 1)
        return carry

    lax.fori_loop(0, PPW // 2, pair_body, 0)
    pltpu.make_async_copy(d_hbm.at[base], drow0, sem0).wait()


def _pipelined_points(d_hbm, drow0, drow1, sem0, sem1, base, process):
    _pipelined_points2(d_hbm, drow0, drow1, sem0, sem1, base,
                       process, process)


def _sc_gather_rows(d_hbm, x_hbm, out_hbm, drow0, drow1, selbuf, rows,
                    outpA, outpB, sem0, sem1, gsem, osemA, osemB):
    wid = lax.axis_index("s") * 2 + lax.axis_index("c")
    base = wid * PPW
    colbase = (base // N) * N
    iota16 = lax.iota(jnp.int32, 16)
    nchunks = N // 16

    def make_process(outp, osem):
        def process(drow, p):
            _sc_compact(drow, selbuf, colbase, iota16, nchunks)
            pltpu.async_copy(x_hbm.at[selbuf.at[pl.ds(0, KNN)]], rows,
                             gsem).wait()

            @pl.when(p >= 2)
            def _():
                pltpu.make_async_copy(outp, out_hbm.at[pl.ds(0, KNN)],
                                      osem).wait()

            def copy_body(k, c):
                outp[k] = rows[k, pl.ds(0, 16)]
                return c

            lax.fori_loop(0, KNN, copy_body, 0)
            pltpu.async_copy(outp,
                             out_hbm.at[pl.ds((base + p) * KNN, KNN)],
                             osem)
        return process

    procA = make_process(outpA, osemA)
    procB = make_process(outpB, osemB)
    _pipelined_points2(d_hbm, drow0, drow1, sem0, sem1, base, procA, procB)
    pltpu.make_async_copy(outpA, out_hbm.at[pl.ds(0, KNN)], osemA).wait()
    pltpu.make_async_copy(outpB, out_hbm.at[pl.ds(0, KNN)], osemB).wait()


def _run_sc_gather_rows(d_flat, xpad):
    mesh = plsc.VectorSubcoreMesh(core_axis_name="c", subcore_axis_name="s",
                                  num_cores=2, num_subcores=16)
    kfn = pl.kernel(
        _sc_gather_rows,
        out_type=jax.ShapeDtypeStruct((B * N * KNN, 16), jnp.float32),
        mesh=mesh,
        scratch_types=[
            pltpu.VMEM((N,), jnp.float32),
            pltpu.VMEM((N,), jnp.float32),
            pltpu.VMEM((KNN + 16,), jnp.int32),
            pltpu.VMEM((KNN, OPAD), jnp.float32),
            pltpu.VMEM((KNN, 16), jnp.float32),
            pltpu.VMEM((KNN, 16), jnp.float32),
            pltpu.SemaphoreType.DMA,
            pltpu.SemaphoreType.DMA,
            pltpu.SemaphoreType.DMA,
            pltpu.SemaphoreType.DMA,
            pltpu.SemaphoreType.DMA,
        ],
        compiler_params=pltpu.CompilerParams(needs_layout_passes=False),
    )
    return kfn(d_flat, xpad)


def _sc_gather_reduce(d_hbm, u_hbm, out_hbm,
                      drow0, drow1, selbuf, rows, outbuf,
                      sem0, sem1, gsem):
    wid = lax.axis_index("s") * 2 + lax.axis_index("c")
    base = wid * PPW
    colbase = (base // N) * N
    iota16 = lax.iota(jnp.int32, 16)
    nchunks = N // 16

    def process(drow, p):
        _sc_compact(drow, selbuf, colbase, iota16, nchunks)

        pltpu.async_copy(u_hbm.at[selbuf.at[pl.ds(0, KNN)]], rows,
                         gsem).wait()

        ninf = jnp.full((16,), -jnp.inf, jnp.float32)
        pinf = jnp.full((16,), jnp.inf, jnp.float32)
        zero = jnp.zeros((16,), jnp.float32)
        acc0 = (ninf, ninf, ninf, ninf, pinf, pinf, pinf, pinf,
                zero, zero, zero, zero, zero, zero, zero, zero)

        def red_body(k, acc):
            out = list(acc)
            for c in range(4):
                rv = rows[k, pl.ds(c * 16, 16)]
                out[c] = jnp.maximum(out[c], rv)
                out[4 + c] = jnp.minimum(out[4 + c], rv)
                out[8 + c] = out[8 + c] + rv
                out[12 + c] = out[12 + c] + rv * rv
            return tuple(out)

        acc = lax.fori_loop(0, KNN, red_body, acc0)
        for c in range(4):
            outbuf[p, pl.ds(c * 16, 16)] = acc[c]
            outbuf[p, pl.ds(64 + c * 16, 16)] = acc[4 + c]
            outbuf[p, pl.ds(128 + c * 16, 16)] = acc[8 + c]
            outbuf[p, pl.ds(192 + c * 16, 16)] = acc[12 + c]

    _pipelined_points(d_hbm, drow0, drow1, sem0, sem1, base, process)
    pltpu.sync_copy(outbuf, out_hbm.at[pl.ds(base, PPW)])


def _run_sc_gather(d_flat, u_flat, O=64):
    mesh = plsc.VectorSubcoreMesh(core_axis_name="c", subcore_axis_name="s",
                                  num_cores=2, num_subcores=16)
    kfn = pl.kernel(
        _sc_gather_reduce,
        out_type=jax.ShapeDtypeStruct((B * N, 4 * O), jnp.float32),
        mesh=mesh,
        scratch_types=[
            pltpu.VMEM((N,), jnp.float32),
            pltpu.VMEM((N,), jnp.float32),
            pltpu.VMEM((KNN + 16,), jnp.int32),
            pltpu.VMEM((KNN, OPAD), jnp.float32),
            pltpu.VMEM((PPW, 4 * O), jnp.float32),
            pltpu.SemaphoreType.DMA,
            pltpu.SemaphoreType.DMA,
            pltpu.SemaphoreType.DMA,
        ],
        compiler_params=pltpu.CompilerParams(needs_layout_passes=False),
    )
    return kfn(d_flat, u_flat)


PB = 64
NBLK = (B * N) // PB


def _edgey_kernel(xg_ref, xf_ref, w_ref, ymax_ref, ymin_ref,
                  psum_ref, psq_ref):
    C = xf_ref.shape[-1]
    xj = xg_ref[:, 0:C]
    xn = jnp.broadcast_to(xf_ref[...][:, None, :],
                          (PB, KNN, C)).reshape(PB * KNN, C)
    edge = jnp.concatenate([xj - xn, xn], axis=-1)
    y = lax.dot_general(edge, w_ref[...], (((1,), (1,)), ((), ())),
                        preferred_element_type=jnp.float32)
    O = y.shape[-1]
    y3 = y.reshape(PB, KNN, O)
    ymax_ref[...] = jnp.max(y3, axis=1)
    ymin_ref[...] = jnp.min(y3, axis=1)
    psum_ref[...] = jnp.sum(y, axis=0)[None, None]
    psq_ref[...] = jnp.sum(y * y, axis=0)[None, None]


def _run_edgey(xg, x_flat, W):
    C = x_flat.shape[-1]
    O = W.shape[0]
    return pl.pallas_call(
        _edgey_kernel,
        grid=(NBLK,),
        in_specs=[
            pl.BlockSpec((PB * KNN, 16), lambda i: (i, 0)),
            pl.BlockSpec((PB, C), lambda i: (i, 0)),
            pl.BlockSpec(W.shape, lambda i: (0, 0)),
        ],
        out_specs=[
            pl.BlockSpec((PB, O), lambda i: (i, 0)),
            pl.BlockSpec((PB, O), lambda i: (i, 0)),
            pl.BlockSpec((1, 1, O), lambda i: (i, 0, 0)),
            pl.BlockSpec((1, 1, O), lambda i: (i, 0, 0)),
        ],
        out_shape=[
            jax.ShapeDtypeStruct((B * N, O), jnp.float32),
            jax.ShapeDtypeStruct((B * N, O), jnp.float32),
            jax.ShapeDtypeStruct((NBLK, 1, O), jnp.float32),
            jax.ShapeDtypeStruct((NBLK, 1, O), jnp.float32),
        ],
    )(xg, x_flat, W)


def _bn_finalize_kernel(ymax_ref, ymin_ref, psum_ref, psq_ref,
                        g_ref, be_ref, w2_ref, h1_ref, u2_ref, v2_ref):
    denom = float(B * N * KNN)
    mean = jnp.sum(psum_ref[...][:, 0, :], axis=0) / denom
    ey2 = jnp.sum(psq_ref[...][:, 0, :], axis=0) / denom
    var = ey2 - mean * mean
    s = jnp.sqrt(var + EPS)
    g = g_ref[...]
    be = be_ref[...]
    M = jnp.where((g >= 0)[None, :], ymax_ref[...], ymin_ref[...])
    yn = (g[None, :] * (M - mean[None, :])) / s[None, :] + be[None, :]
    h1 = jnp.where(yn >= 0, yn, NEG_SLOPE * yn)
    h1_ref[...] = h1
    C = h1.shape[-1]
    wd = w2_ref[:, :C]
    wc = w2_ref[:, C:]
    u2 = lax.dot_general(h1, wd, (((1,), (1,)), ((), ())),
                         preferred_element_type=jnp.float32)
    O2 = u2.shape[-1]
    u2_ref[...] = jnp.concatenate(
        [u2, jnp.zeros((B * N, OPAD - O2), jnp.float32)], axis=-1)
    v2_ref[...] = lax.dot_general(h1, wc - wd, (((1,), (1,)), ((), ())),
                                  preferred_element_type=jnp.float32)


def _run_bn_finalize(ymax, ymin, psum, psq, g, be, W2):
    O = ymax.shape[-1]
    O2 = W2.shape[0]
    return pl.pallas_call(
        _bn_finalize_kernel,
        out_shape=[
            jax.ShapeDtypeStruct((B * N, O), jnp.float32),
            jax.ShapeDtypeStruct((B * N, OPAD), jnp.float32),
            jax.ShapeDtypeStruct((B * N, O2), jnp.float32),
        ],
    )(ymax, ymin, psum, psq, g, be, W2)


def _bn_stats_kernel(gstats_ref, vv_ref, mean_ref, rstd_ref):
    O = vv_ref.shape[-1]
    gsum = gstats_ref[:, 2 * O:3 * O]
    gsq = gstats_ref[:, 3 * O:4 * O]
    v = vv_ref[...]
    denom = float(B * N * KNN)
    mean = (jnp.sum(gsum, axis=0) + KNN * jnp.sum(v, axis=0)) / denom
    ey2 = (jnp.sum(gsq, axis=0) + 2.0 * jnp.sum(v * gsum, axis=0)
           + KNN * jnp.sum(v * v, axis=0)) / denom
    var = ey2 - mean * mean
    mean_ref[...] = mean
    rstd_ref[...] = lax.rsqrt(var + EPS)


def _run_bn_stats(gstats, v_flat):
    O = v_flat.shape[-1]
    return pl.pallas_call(
        _bn_stats_kernel,
        out_shape=[
            jax.ShapeDtypeStruct((O,), jnp.float32),
            jax.ShapeDtypeStruct((O,), jnp.float32),
        ],
    )(gstats, v_flat)


def _final_kernel(h1_ref, gstats_ref, v2_ref, mean_ref, rstd_ref,
                  g_ref, be_ref, wf_ref, bf_ref, out_ref):
    O = v2_ref.shape[-1]
    gmax = gstats_ref[0, :, 0:O]
    gmin = gstats_ref[0, :, O:2 * O]
    v = v2_ref[0]
    g = g_ref[...]
    be = be_ref[...]
    M = jnp.where((g >= 0)[None, :], gmax + v, gmin + v)
    yn = g[None, :] * (M - mean_ref[...][None, :]) * rstd_ref[...][None, :] \
        + be[None, :]
    h2 = jnp.where(yn >= 0, yn, NEG_SLOPE * yn)
    hcat = jnp.concatenate([h1_ref[0], h2], axis=-1)
    o = jnp.dot(hcat, wf_ref[...], preferred_element_type=jnp.float32)
    o = o + bf_ref[...][None, :]
    out_ref[0, 0] = jnp.max(o, axis=0)


def _run_final(h1, gstats2, v2, mean2, rstd2, g2, be2, Wf, bf):
    O = v2.shape[-1]
    Z = Wf.shape[1]
    return pl.pallas_call(
        _final_kernel,
        grid=(B,),
        in_specs=[
            pl.BlockSpec((1, N, O), lambda b: (b, 0, 0)),
            pl.BlockSpec((1, N, 4 * O), lambda b: (b, 0, 0)),
            pl.BlockSpec((1, N, O), lambda b: (b, 0, 0)),
            pl.BlockSpec((O,), lambda b: (0,)),
            pl.BlockSpec((O,), lambda b: (0,)),
            pl.BlockSpec((O,), lambda b: (0,)),
            pl.BlockSpec((O,), lambda b: (0,)),
            pl.BlockSpec(Wf.shape, lambda b: (0, 0)),
            pl.BlockSpec((Z,), lambda b: (0,)),
        ],
        out_specs=pl.BlockSpec((1, 1, Z), lambda b: (b, 0, 0)),
        out_shape=jax.ShapeDtypeStruct((B, 1, Z), jnp.float32),
    )(h1, gstats2, v2, mean2, rstd2, g2, be2, Wf, bf)


def kernel(x, W1, b1, g1, be1, W2, b2, g2, be2, Wf, bf):
    D1, xpad = _run_dist_x(x)
    xg = _run_sc_gather_rows(D1.reshape(B * N, N),
                             xpad.reshape(B * N, OPAD))
    ymax, ymin, psum, psq = _run_edgey(xg, x.reshape(B * N, -1), W1)
    h1f, u2f, v2f = _run_bn_finalize(ymax, ymin, psum, psq, g1, be1, W2)
    h1 = h1f.reshape(B, N, -1)
    D2 = _run_dist(h1)
    gstats2 = _run_sc_gather(D2.reshape(B * N, N), u2f)
    mean2, rstd2 = _run_bn_stats(gstats2, v2f)
    out = _run_final(h1, gstats2.reshape(B, N, -1), v2f.reshape(B, N, -1),
                     mean2, rstd2, g2, be2, Wf, bf)
    return out[:, 0, :]
